# baseline (speedup 1.0000x reference)
import dataclasses
import functools

import jax
import jax.numpy as jnp
from jax import lax
from jax.experimental import pallas as pl
from jax.experimental.pallas import tpu as pltpu
from jax.experimental.pallas import tpu_sc as plsc

N = 10000
D = 128
E = 320000
NC = 2
NS = 16
NW = NC * NS
EDGES_PER_W = E // NW
BATCH = 80
NBATCH = EDGES_PER_W // BATCH
NBUF = 3
PIPE_ITERS = -(-(NBATCH + NBUF) // NBUF)
ZCHUNK = 1000
ZTILES = N // ZCHUNK
CNT_W = 16


def _sc_aggregate(h, pk2d, zsum):
  mesh = plsc.VectorSubcoreMesh(core_axis_name="c", subcore_axis_name="s")

  def body(h_hbm, pk_hbm, zsum_hbm, sum_out,
           pk_v, sdec, ddec, rows_v, acc_sp, *sems):
    c = lax.axis_index("c")
    s = lax.axis_index("s")
    w = c * NS + s
    rows = pl.ds(s * ZCHUNK, ZCHUNK)

    @pl.when(s < ZTILES)
    def _():
      pltpu.sync_copy(zsum_hbm.at[rows], acc_sp.at[rows])

    pltpu.sync_copy(pk_hbm.at[w], pk_v)
    plsc.subcore_barrier()

    mask = jnp.int32(0xFFFF)
    gsems = sems[:NBUF]
    ssems = sems[NBUF:]

    def start_gather(b, j):
      for k in range(BATCH // 16):
        sl = pl.ds(k * 16, 16)
        pk = pk_v[j, sl]
        sdec[b, sl] = jnp.bitwise_and(pk, mask)
        ddec[b, sl] = lax.shift_right_logical(pk, 16)
      pltpu.async_copy(h_hbm.at[sdec.at[b]], rows_v.at[b], gsems[b])

    def wait_gather_start_scatter(b):
      pltpu.make_async_copy(h_hbm.at[sdec.at[b]], rows_v.at[b],
                            gsems[b]).wait()
      pltpu.async_copy(rows_v.at[b], acc_sp.at[ddec.at[b]], ssems[b],
                       add=True)

    def wait_scatter(b):
      pltpu.make_async_copy(rows_v.at[b], acc_sp.at[ddec.at[b]],
                            ssems[b]).wait()

    @pl.loop(0, PIPE_ITERS)
    def _(i):
      for b in range(NBUF):
        j = i * NBUF + b
        jg = j - 1
        jc = j - NBUF
        bg = (b - 1) % NBUF

        @pl.when(jnp.logical_and(jg >= 0, jg < NBATCH))
        def _():
          wait_gather_start_scatter(bg)

        @pl.when(jnp.logical_and(jc >= 0, jc < NBATCH))
        def _():
          wait_scatter(b)

        @pl.when(j < NBATCH)
        def _():
          start_gather(b, j)

    plsc.subcore_barrier()

    @pl.when(s < ZTILES)
    def _():
      pltpu.sync_copy(acc_sp.at[rows], sum_out.at[c, rows])

  f = pl.kernel(
      body,
      out_type=jax.ShapeDtypeStruct((NC, N, D), jnp.float32),
      mesh=mesh,
      scratch_types=[
          pltpu.VMEM((NBATCH, BATCH), jnp.int32),
          pltpu.VMEM((NBUF, BATCH), jnp.int32),
          pltpu.VMEM((NBUF, BATCH), jnp.int32),
          pltpu.VMEM((NBUF, BATCH, D), jnp.float32),
          pltpu.VMEM_SHARED((N, D), jnp.float32),
      ] + [pltpu.SemaphoreType.DMA] * (2 * NBUF),
  )
  return f(h, pk2d, zsum)


def _sc_counts(dst16):
  mesh = plsc.VectorSubcoreMesh(core_axis_name="c", subcore_axis_name="s")
  NVEC = E // NW // 16

  def body(dst_hbm, cnt_out, dst_v, hist_v):
    c = lax.axis_index("c")
    s = lax.axis_index("s")
    w = c * NS + s

    @pl.loop(0, N // 16)
    def _(i):
      hist_v[pl.ds(i * 16, 16)] = jnp.zeros((16,), jnp.float32)

    pltpu.sync_copy(dst_hbm.at[w], dst_v)
    ones = jnp.full((16,), 1.0, jnp.float32)

    @pl.loop(0, NVEC)
    def _(j):
      plsc.addupdate_scatter(hist_v, [dst_v[j, :]], ones)

    pltpu.sync_copy(hist_v, cnt_out.at[w, 0])

  cp = pltpu.CompilerParams()
  if "needs_layout_passes" in pltpu.CompilerParams.__dataclass_fields__:
    cp = dataclasses.replace(cp, needs_layout_passes=False)
  f = pl.kernel(
      body,
      out_type=jax.ShapeDtypeStruct((NW, 1, N), jnp.float32),
      mesh=mesh,
      scratch_types=[
          pltpu.VMEM((NVEC, 16), jnp.int32),
          pltpu.VMEM((N,), jnp.float32),
      ],
      compiler_params=cp,
  )
  return f(dst16)


_TC_R = 2000


def _tc_project(x, Wp, bp):
  def body(x_ref, w_ref, b_ref, o_ref):
    o_ref[...] = jnp.maximum(
        lax.dot_general(x_ref[...], w_ref[...], (((1,), (1,)), ((), ())),
                        preferred_element_type=jnp.float32) + b_ref[...],
        0.0)

  return pl.pallas_call(
      body,
      grid=(N // _TC_R,),
      in_specs=[
          pl.BlockSpec((_TC_R, D), lambda i: (i, 0)),
          pl.BlockSpec((D, D), lambda i: (0, 0)),
          pl.BlockSpec((1, D), lambda i: (0, 0)),
      ],
      out_specs=pl.BlockSpec((_TC_R, D), lambda i: (i, 0)),
      out_shape=jax.ShapeDtypeStruct((N, D), jnp.float32),
  )(x, Wp, bp.reshape(1, D))


def _tc_mid(parts, cnts, x, Wl0, bl0, Wr0, Wp1, bp1):
  def body(p_ref, c_ref, x_ref, wl_ref, bl_ref, wr_ref, wp_ref, bp_ref,
           x1_ref, h1_ref):
    ssum = p_ref[0] + p_ref[1]
    cnt = jnp.sum(c_ref[...], axis=1, keepdims=True)
    mean = ssum / jnp.maximum(cnt, 1.0)
    x1 = (lax.dot_general(mean, wl_ref[...], (((1,), (1,)), ((), ())),
                          preferred_element_type=jnp.float32)
          + bl_ref[...]
          + lax.dot_general(x_ref[...], wr_ref[...], (((1,), (1,)), ((), ())),
                            preferred_element_type=jnp.float32))
    x1 = jnp.maximum(x1, 0.0)
    x1_ref[...] = x1
    h1_ref[...] = jnp.maximum(
        lax.dot_general(x1, wp_ref[...], (((1,), (1,)), ((), ())),
                        preferred_element_type=jnp.float32) + bp_ref[...],
        0.0)

  return pl.pallas_call(
      body,
      grid=(N // _TC_R,),
      in_specs=[
          pl.BlockSpec((NC, _TC_R, D), lambda i: (0, i, 0)),
          pl.BlockSpec((_TC_R, NW), lambda i: (i, 0)),
          pl.BlockSpec((_TC_R, D), lambda i: (i, 0)),
          pl.BlockSpec((D, D), lambda i: (0, 0)),
          pl.BlockSpec((1, D), lambda i: (0, 0)),
          pl.BlockSpec((D, D), lambda i: (0, 0)),
          pl.BlockSpec((D, D), lambda i: (0, 0)),
          pl.BlockSpec((1, D), lambda i: (0, 0)),
      ],
      out_specs=[
          pl.BlockSpec((_TC_R, D), lambda i: (i, 0)),
          pl.BlockSpec((_TC_R, D), lambda i: (i, 0)),
      ],
      out_shape=[
          jax.ShapeDtypeStruct((N, D), jnp.float32),
          jax.ShapeDtypeStruct((N, D), jnp.float32),
      ],
  )(parts, cnts, x, Wl0, bl0.reshape(1, D), Wr0, Wp1, bp1.reshape(1, D))


def _tc_final(parts, cnts, x1, Wl1, bl1, Wr1):
  def body(p_ref, c_ref, x_ref, wl_ref, bl_ref, wr_ref, o_ref):
    ssum = p_ref[0] + p_ref[1]
    cnt = jnp.sum(c_ref[...], axis=1, keepdims=True)
    mean = ssum / jnp.maximum(cnt, 1.0)
    o_ref[...] = (lax.dot_general(mean, wl_ref[...], (((1,), (1,)), ((), ())),
                                  preferred_element_type=jnp.float32)
                  + bl_ref[...]
                  + lax.dot_general(x_ref[...], wr_ref[...],
                                    (((1,), (1,)), ((), ())),
                                    preferred_element_type=jnp.float32))

  return pl.pallas_call(
      body,
      grid=(N // _TC_R,),
      in_specs=[
          pl.BlockSpec((NC, _TC_R, D), lambda i: (0, i, 0)),
          pl.BlockSpec((_TC_R, NW), lambda i: (i, 0)),
          pl.BlockSpec((_TC_R, D), lambda i: (i, 0)),
          pl.BlockSpec((D, D), lambda i: (0, 0)),
          pl.BlockSpec((1, D), lambda i: (0, 0)),
          pl.BlockSpec((D, D), lambda i: (0, 0)),
      ],
      out_specs=pl.BlockSpec((_TC_R, D), lambda i: (i, 0)),
      out_shape=jax.ShapeDtypeStruct((N, D), jnp.float32),
  )(parts, cnts, x1, Wl1, bl1.reshape(1, D), Wr1)


def kernel(x, edge_index, Wp0, bp0, Wl0, bl0, Wr0, Wp1, bp1, Wl1, bl1, Wr1):
  src = edge_index[0]
  dst = edge_index[1]
  pk2d = jnp.bitwise_or(src, jnp.left_shift(dst, 16)).reshape(
      NW, NBATCH, BATCH)
  dst16 = dst.reshape(NW, E // NW // 16, 16)
  zsum = jnp.zeros((N, D), jnp.float32)

  cnts = _sc_counts(dst16)
  cnts_t = cnts.reshape(NW, N).T
  h0 = _tc_project(x, Wp0, bp0)
  sums0 = _sc_aggregate(h0, pk2d, zsum)
  x1, h1 = _tc_mid(sums0, cnts_t, x, Wl0, bl0, Wr0, Wp1, bp1)
  sums1 = _sc_aggregate(h1, pk2d, zsum)
  return _tc_final(sums1, cnts_t, x1, Wl1, bl1, Wr1)


# speedup vs baseline: 1.4676x; 1.4676x over previous
import dataclasses
import functools

import jax
import jax.numpy as jnp
from jax import lax
from jax.experimental import pallas as pl
from jax.experimental.pallas import tpu as pltpu
from jax.experimental.pallas import tpu_sc as plsc

N = 10000
D = 128
E = 320000
NC = 2
NS = 16
NW = NC * NS
EDGES_PER_W = E // NW
BATCH = 80
NBATCH = EDGES_PER_W // BATCH
NBUF = 3
PIPE_ITERS = -(-(NBATCH + NBUF) // NBUF)
ZCHUNK = 1000
ZTILES = N // ZCHUNK
CNT_W = 16


def _sc_aggregate(h, pk2d, zsum):
  mesh = plsc.VectorSubcoreMesh(core_axis_name="c", subcore_axis_name="s")

  def body(h_hbm, pk_hbm, zsum_hbm, sum_out,
           pk_v, sdec, ddec, rows_v, acc_sp, *sems):
    c = lax.axis_index("c")
    s = lax.axis_index("s")
    w = c * NS + s
    rows = pl.ds(s * ZCHUNK, ZCHUNK)

    @pl.when(s < ZTILES)
    def _():
      pltpu.sync_copy(zsum_hbm.at[rows], acc_sp.at[rows])

    pltpu.sync_copy(pk_hbm.at[w], pk_v)
    plsc.subcore_barrier()

    mask = jnp.int32(0xFFFF)

    def start(b, j):
      for k in range(BATCH // 16):
        sl = pl.ds(k * 16, 16)
        pk = pk_v[j, sl]
        sdec[b, sl] = jnp.bitwise_and(pk, mask)
        ddec[b, sl] = lax.shift_right_logical(pk, 16)
      pltpu.async_copy(h_hbm.at[sdec.at[b]], rows_v.at[b], sems[b])

    def finish(b):
      pltpu.make_async_copy(h_hbm.at[sdec.at[b]], rows_v.at[b],
                            sems[b]).wait()
      pltpu.sync_copy(rows_v.at[b], acc_sp.at[ddec.at[b]], add=True)

    @pl.loop(0, PIPE_ITERS)
    def _(i):
      for b in range(NBUF):
        jf = i * NBUF + b - NBUF
        js = i * NBUF + b

        @pl.when(jnp.logical_and(jf >= 0, jf < NBATCH))
        def _():
          finish(b)

        @pl.when(js < NBATCH)
        def _():
          start(b, js)

    plsc.subcore_barrier()

    @pl.when(s < ZTILES)
    def _():
      pltpu.sync_copy(acc_sp.at[rows], sum_out.at[c, rows])

  f = pl.kernel(
      body,
      out_type=jax.ShapeDtypeStruct((NC, N, D), jnp.float32),
      mesh=mesh,
      scratch_types=[
          pltpu.VMEM((NBATCH, BATCH), jnp.int32),
          pltpu.VMEM((NBUF, BATCH), jnp.int32),
          pltpu.VMEM((NBUF, BATCH), jnp.int32),
          pltpu.VMEM((NBUF, BATCH, D), jnp.float32),
          pltpu.VMEM_SHARED((N, D), jnp.float32),
      ] + [pltpu.SemaphoreType.DMA] * NBUF,
  )
  return f(h, pk2d, zsum)


def _sc_counts(dst16):
  mesh = plsc.VectorSubcoreMesh(core_axis_name="c", subcore_axis_name="s")
  NVEC = E // NW // 16

  def body(dst_hbm, cnt_out, dst_v, hist_v):
    c = lax.axis_index("c")
    s = lax.axis_index("s")
    w = c * NS + s

    @pl.loop(0, N // 16)
    def _(i):
      hist_v[pl.ds(i * 16, 16)] = jnp.zeros((16,), jnp.float32)

    pltpu.sync_copy(dst_hbm.at[w], dst_v)
    ones = jnp.full((16,), 1.0, jnp.float32)

    @pl.loop(0, NVEC)
    def _(j):
      plsc.addupdate_scatter(hist_v, [dst_v[j, :]], ones)

    pltpu.sync_copy(hist_v, cnt_out.at[w, 0])

  cp = pltpu.CompilerParams()
  if "needs_layout_passes" in pltpu.CompilerParams.__dataclass_fields__:
    cp = dataclasses.replace(cp, needs_layout_passes=False)
  f = pl.kernel(
      body,
      out_type=jax.ShapeDtypeStruct((NW, 1, N), jnp.float32),
      mesh=mesh,
      scratch_types=[
          pltpu.VMEM((NVEC, 16), jnp.int32),
          pltpu.VMEM((N,), jnp.float32),
      ],
      compiler_params=cp,
  )
  return f(dst16)


_TC_R = 2000


def _tc_project(x, Wp, bp):
  def body(x_ref, w_ref, b_ref, o_ref):
    o_ref[...] = jnp.maximum(
        lax.dot_general(x_ref[...], w_ref[...], (((1,), (1,)), ((), ())),
                        preferred_element_type=jnp.float32) + b_ref[...],
        0.0)

  return pl.pallas_call(
      body,
      grid=(N // _TC_R,),
      in_specs=[
          pl.BlockSpec((_TC_R, D), lambda i: (i, 0)),
          pl.BlockSpec((D, D), lambda i: (0, 0)),
          pl.BlockSpec((1, D), lambda i: (0, 0)),
      ],
      out_specs=pl.BlockSpec((_TC_R, D), lambda i: (i, 0)),
      out_shape=jax.ShapeDtypeStruct((N, D), jnp.float32),
  )(x, Wp, bp.reshape(1, D))


def _tc_mid(parts, cnts, x, Wl0, bl0, Wr0, Wp1, bp1):
  def body(p_ref, c_ref, x_ref, wl_ref, bl_ref, wr_ref, wp_ref, bp_ref,
           x1_ref, h1_ref):
    ssum = p_ref[0] + p_ref[1]
    cnt = jnp.sum(c_ref[...], axis=1, keepdims=True)
    mean = ssum / jnp.maximum(cnt, 1.0)
    x1 = (lax.dot_general(mean, wl_ref[...], (((1,), (1,)), ((), ())),
                          preferred_element_type=jnp.float32)
          + bl_ref[...]
          + lax.dot_general(x_ref[...], wr_ref[...], (((1,), (1,)), ((), ())),
                            preferred_element_type=jnp.float32))
    x1 = jnp.maximum(x1, 0.0)
    x1_ref[...] = x1
    h1_ref[...] = jnp.maximum(
        lax.dot_general(x1, wp_ref[...], (((1,), (1,)), ((), ())),
                        preferred_element_type=jnp.float32) + bp_ref[...],
        0.0)

  return pl.pallas_call(
      body,
      grid=(N // _TC_R,),
      in_specs=[
          pl.BlockSpec((NC, _TC_R, D), lambda i: (0, i, 0)),
          pl.BlockSpec((_TC_R, NW), lambda i: (i, 0)),
          pl.BlockSpec((_TC_R, D), lambda i: (i, 0)),
          pl.BlockSpec((D, D), lambda i: (0, 0)),
          pl.BlockSpec((1, D), lambda i: (0, 0)),
          pl.BlockSpec((D, D), lambda i: (0, 0)),
          pl.BlockSpec((D, D), lambda i: (0, 0)),
          pl.BlockSpec((1, D), lambda i: (0, 0)),
      ],
      out_specs=[
          pl.BlockSpec((_TC_R, D), lambda i: (i, 0)),
          pl.BlockSpec((_TC_R, D), lambda i: (i, 0)),
      ],
      out_shape=[
          jax.ShapeDtypeStruct((N, D), jnp.float32),
          jax.ShapeDtypeStruct((N, D), jnp.float32),
      ],
  )(parts, cnts, x, Wl0, bl0.reshape(1, D), Wr0, Wp1, bp1.reshape(1, D))


def _tc_final(parts, cnts, x1, Wl1, bl1, Wr1):
  def body(p_ref, c_ref, x_ref, wl_ref, bl_ref, wr_ref, o_ref):
    ssum = p_ref[0] + p_ref[1]
    cnt = jnp.sum(c_ref[...], axis=1, keepdims=True)
    mean = ssum / jnp.maximum(cnt, 1.0)
    o_ref[...] = (lax.dot_general(mean, wl_ref[...], (((1,), (1,)), ((), ())),
                                  preferred_element_type=jnp.float32)
                  + bl_ref[...]
                  + lax.dot_general(x_ref[...], wr_ref[...],
                                    (((1,), (1,)), ((), ())),
                                    preferred_element_type=jnp.float32))

  return pl.pallas_call(
      body,
      grid=(N // _TC_R,),
      in_specs=[
          pl.BlockSpec((NC, _TC_R, D), lambda i: (0, i, 0)),
          pl.BlockSpec((_TC_R, NW), lambda i: (i, 0)),
          pl.BlockSpec((_TC_R, D), lambda i: (i, 0)),
          pl.BlockSpec((D, D), lambda i: (0, 0)),
          pl.BlockSpec((1, D), lambda i: (0, 0)),
          pl.BlockSpec((D, D), lambda i: (0, 0)),
      ],
      out_specs=pl.BlockSpec((_TC_R, D), lambda i: (i, 0)),
      out_shape=jax.ShapeDtypeStruct((N, D), jnp.float32),
  )(parts, cnts, x1, Wl1, bl1.reshape(1, D), Wr1)


def kernel(x, edge_index, Wp0, bp0, Wl0, bl0, Wr0, Wp1, bp1, Wl1, bl1, Wr1):
  src = edge_index[0]
  dst = edge_index[1]
  pk2d = jnp.bitwise_or(src, jnp.left_shift(dst, 16)).reshape(
      NW, NBATCH, BATCH)
  dst16 = dst.reshape(NW, E // NW // 16, 16)
  zsum = jnp.zeros((N, D), jnp.float32)

  cnts = _sc_counts(dst16)
  cnts_t = cnts.reshape(NW, N).T
  h0 = _tc_project(x, Wp0, bp0)
  sums0 = _sc_aggregate(h0, pk2d, zsum)
  x1, h1 = _tc_mid(sums0, cnts_t, x, Wl0, bl0, Wr0, Wp1, bp1)
  sums1 = _sc_aggregate(h1, pk2d, zsum)
  return _tc_final(sums1, cnts_t, x1, Wl1, bl1, Wr1)
